# Initial kernel scaffold; baseline (speedup 1.0000x reference)
import jax
import jax.numpy as jnp
from jax.experimental import pallas as pl


def kernel(node_features, edge_index, W, attn, bias):
    raise NotImplementedError("write your pallas kernel here")


# speedup vs baseline: 17.3178x; 17.3178x over previous
import functools

import jax
import jax.numpy as jnp
from jax import lax
from jax.experimental import pallas as pl
from jax.experimental.pallas import tpu as pltpu
from jax.experimental.pallas import tpu_sc as plsc

NC = 2
NS = 16
LANES = 16
BATCH = 80
ROWBLK = 2000


def _pre_body(x_ref, w_ref, a1_ref, a2_ref, h_ref, as_ref, ad_ref):
    h = lax.dot_general(
        x_ref[...], w_ref[...], (((1,), (0,)), ((), ())),
        precision=lax.Precision.HIGHEST, preferred_element_type=jnp.float32)
    h_ref[...] = h
    as_ref[...] = jnp.sum(h * a1_ref[...], axis=1, keepdims=True)
    ad_ref[...] = jnp.sum(h * a2_ref[...], axis=1, keepdims=True)


def _post_body(acc_ref, s_ref, b_ref, o_ref):
    acc = acc_ref[0] + acc_ref[1]
    s = s_ref[0] + s_ref[1]
    ok = s > 0.0
    safe = jnp.where(ok, s, 1.0)
    o_ref[...] = jnp.where(ok, acc / safe, 0.0) + b_ref[...]


def _sc_body(src_hbm, dst_hbm, asrc_hbm, adst_hbm, h_hbm,
             acc_out, s_out,
             asrc_v, adst_v, sidx_v, didx_v, rows_v, w_v, stage_v, svec_v,
             acc_sh, s_sh, sem):
    n = asrc_v.shape[0]
    d_out = rows_v.shape[1]
    sub = stage_v.shape[0]
    cid = lax.axis_index("c")
    tid = lax.axis_index("s")
    gtid = cid * NS + tid
    e_total = src_hbm.shape[0]
    ept = e_total // (NC * NS)
    nb = ept // BATCH
    ebase = gtid * ept

    pltpu.sync_copy(asrc_hbm, asrc_v)
    pltpu.sync_copy(adst_hbm, adst_v)

    ch = (n // NS) & ~7
    rem = n - NS * ch
    zbase = tid * ch
    z16 = jnp.zeros((LANES,), jnp.float32)

    def zrow_body(j, c):
        for d in range(d_out // LANES):
            stage_v[j, pl.ds(d * LANES, LANES)] = z16
        return c

    lax.fori_loop(0, sub, zrow_body, 0)

    def zs_body(i, c):
        svec_v[pl.ds(i * LANES, LANES)] = z16
        return c

    lax.fori_loop(0, ch // LANES, zs_body, 0)

    for k in range(ch // sub):
        pltpu.sync_copy(stage_v, acc_sh.at[pl.ds(zbase + k * sub, sub)])
    pltpu.sync_copy(svec_v, s_sh.at[pl.ds(zbase, ch)])

    @pl.when(tid == NS - 1)
    def _():
        pltpu.sync_copy(stage_v.at[pl.ds(0, rem)],
                        acc_sh.at[pl.ds(NS * ch, rem)])
        pltpu.sync_copy(svec_v.at[pl.ds(0, rem)],
                        s_sh.at[pl.ds(NS * ch, rem)])

    plsc.subcore_barrier()

    def batch_body(b, carry):
        off = ebase + b * BATCH
        pltpu.sync_copy(src_hbm.at[pl.ds(off, BATCH)], sidx_v)
        pltpu.sync_copy(dst_hbm.at[pl.ds(off, BATCH)], didx_v)
        gather = pltpu.async_copy(h_hbm.at[sidx_v], rows_v, sem)
        for i in range(BATCH // LANES):
            si = sidx_v[pl.ds(i * LANES, LANES)]
            di = didx_v[pl.ds(i * LANES, LANES)]
            e = plsc.load_gather(asrc_v, [si]) + plsc.load_gather(adst_v, [di])
            e = jnp.where(e >= 0.0, e, 0.2 * e)
            w_v[pl.ds(i * LANES, LANES)] = jnp.exp(e)
        gather.wait()

        def scale_body(i, c):
            w16 = w_v[pl.ds(i * LANES, LANES)]
            base = i * LANES
            for j in range(LANES):
                ws = w16[j]
                for d in range(d_out // LANES):
                    sl = rows_v[base + j, pl.ds(d * LANES, LANES)]
                    rows_v[base + j, pl.ds(d * LANES, LANES)] = sl * ws
            return c

        lax.fori_loop(0, BATCH // LANES, scale_body, 0)
        pltpu.sync_copy(rows_v, acc_sh.at[didx_v], add=True)
        pltpu.sync_copy(w_v, s_sh.at[didx_v], add=True)
        return carry

    lax.fori_loop(0, nb, batch_body, 0)
    plsc.subcore_barrier()

    for k in range(ch // sub):
        pltpu.sync_copy(acc_sh.at[pl.ds(zbase + k * sub, sub)], stage_v)
        pltpu.sync_copy(stage_v, acc_out.at[cid, pl.ds(zbase + k * sub, sub)])
    pltpu.sync_copy(s_sh.at[pl.ds(zbase, ch)], svec_v)
    pltpu.sync_copy(svec_v, s_out.at[pl.ds(cid * n + zbase, ch)])

    @pl.when(tid == NS - 1)
    def _():
        pltpu.sync_copy(acc_sh.at[pl.ds(NS * ch, rem)], stage_v.at[pl.ds(0, rem)])
        pltpu.sync_copy(stage_v.at[pl.ds(0, rem)],
                        acc_out.at[cid, pl.ds(NS * ch, rem)])
        pltpu.sync_copy(s_sh.at[pl.ds(NS * ch, rem)], svec_v.at[pl.ds(0, rem)])
        pltpu.sync_copy(svec_v.at[pl.ds(0, rem)],
                        s_out.at[pl.ds(cid * n + NS * ch, rem)])


def kernel(node_features, edge_index, W, attn, bias):
    n, d_in = node_features.shape
    d_out = W.shape[1]
    e_total = edge_index.shape[1]

    a1 = attn[:d_out].reshape(1, d_out)
    a2 = attn[d_out:].reshape(1, d_out)

    h, alpha_src, alpha_dst = pl.pallas_call(
        _pre_body,
        grid=(n // ROWBLK,),
        in_specs=[
            pl.BlockSpec((ROWBLK, d_in), lambda i: (i, 0)),
            pl.BlockSpec((d_in, d_out), lambda i: (0, 0)),
            pl.BlockSpec((1, d_out), lambda i: (0, 0)),
            pl.BlockSpec((1, d_out), lambda i: (0, 0)),
        ],
        out_specs=[
            pl.BlockSpec((ROWBLK, d_out), lambda i: (i, 0)),
            pl.BlockSpec((ROWBLK, 1), lambda i: (i, 0)),
            pl.BlockSpec((ROWBLK, 1), lambda i: (i, 0)),
        ],
        out_shape=[
            jax.ShapeDtypeStruct((n, d_out), jnp.float32),
            jax.ShapeDtypeStruct((n, 1), jnp.float32),
            jax.ShapeDtypeStruct((n, 1), jnp.float32),
        ],
    )(node_features, W, a1, a2)

    src = edge_index[0]
    dst = edge_index[1]

    mesh = plsc.VectorSubcoreMesh(
        core_axis_name="c", subcore_axis_name="s",
        num_cores=NC, num_subcores=NS)
    acc, s = pl.kernel(
        _sc_body,
        out_type=[
            jax.ShapeDtypeStruct((NC, n, d_out), jnp.float32),
            jax.ShapeDtypeStruct((NC * n,), jnp.float32),
        ],
        mesh=mesh,
        compiler_params=pltpu.CompilerParams(needs_layout_passes=False),
        scratch_types=[
            pltpu.VMEM((n,), jnp.float32),
            pltpu.VMEM((n,), jnp.float32),
            pltpu.VMEM((BATCH,), jnp.int32),
            pltpu.VMEM((BATCH,), jnp.int32),
            pltpu.VMEM((BATCH, d_out), jnp.float32),
            pltpu.VMEM((BATCH,), jnp.float32),
            pltpu.VMEM((48, d_out), jnp.float32),
            pltpu.VMEM(((n // NS) & ~7,), jnp.float32),
            pltpu.VMEM_SHARED((n, d_out), jnp.float32),
            pltpu.VMEM_SHARED((n,), jnp.float32),
            pltpu.SemaphoreType.DMA,
        ],
    )(src, dst, alpha_src.reshape(n), alpha_dst.reshape(n), h)

    out = pl.pallas_call(
        _post_body,
        grid=(n // ROWBLK,),
        in_specs=[
            pl.BlockSpec((NC, ROWBLK, d_out), lambda i: (0, i, 0)),
            pl.BlockSpec((NC, ROWBLK, 1), lambda i: (0, i, 0)),
            pl.BlockSpec((1, d_out), lambda i: (0, 0)),
        ],
        out_specs=pl.BlockSpec((ROWBLK, d_out), lambda i: (i, 0)),
        out_shape=jax.ShapeDtypeStruct((n, d_out), jnp.float32),
    )(acc, s.reshape(NC, n, 1), bias.reshape(1, d_out))
    return out


# speedup vs baseline: 29.5872x; 1.7085x over previous
import functools

import jax
import jax.numpy as jnp
from jax import lax
from jax.experimental import pallas as pl
from jax.experimental.pallas import tpu as pltpu
from jax.experimental.pallas import tpu_sc as plsc

NC = 2
NS = 16
LANES = 16
BATCH = 96
ROWBLK = 2000


def _pre_body(x_ref, w_ref, a1_ref, a2_ref, h_ref, as_ref, ad_ref):
    h = lax.dot_general(
        x_ref[...], w_ref[...], (((1,), (0,)), ((), ())),
        precision=lax.Precision.HIGHEST, preferred_element_type=jnp.float32)
    h_ref[...] = h
    as_ref[...] = jnp.sum(h * a1_ref[...], axis=1, keepdims=True)
    ad_ref[...] = jnp.sum(h * a2_ref[...], axis=1, keepdims=True)


def _post_body(acc_ref, s_ref, b_ref, o_ref):
    acc = acc_ref[0] + acc_ref[1]
    s = s_ref[0] + s_ref[1]
    ok = s > 0.0
    safe = jnp.where(ok, s, 1.0)
    o_ref[...] = jnp.where(ok, acc / safe, 0.0) + b_ref[...]


def _sc_body(src_hbm, dst_hbm, asrc_hbm, adst_hbm, h_hbm,
             acc_out, s_out,
             asrc_v, adst_v, sidx2, didx2, dsc2, rows2, w2, tsidx, tdidx,
             acc_sh, s_sh,
             isem0, isem1, gsem0, gsem1, ssem0, ssem1):
    n = asrc_v.shape[0]
    d_out = rows2.shape[2]
    b = rows2.shape[1]
    nl = b // LANES
    cid = lax.axis_index("c")
    tid = lax.axis_index("s")
    gtid = cid * NS + tid
    e_total = src_hbm.shape[0]
    ept = e_total // (NC * NS)
    nfull = ept // b
    tail = ept - nfull * b
    ebase = gtid * ept
    isem = (isem0, isem1)
    gsem = (gsem0, gsem1)
    ssem = (ssem0, ssem1)

    ch = (n // NS) & ~7
    rem = n - NS * ch
    zbase = tid * ch
    z16 = jnp.zeros((LANES,), jnp.float32)

    def zrow_body(j, c):
        for d in range(d_out // LANES):
            rows2[0, j, pl.ds(d * LANES, LANES)] = z16
        return c

    lax.fori_loop(0, b, zrow_body, 0)

    def zs_body(i, c):
        asrc_v[pl.ds(i * LANES, LANES)] = z16
        return c

    lax.fori_loop(0, ch // LANES, zs_body, 0)

    nck = ch // b
    for k in range(nck):
        pltpu.sync_copy(rows2.at[0], acc_sh.at[pl.ds(zbase + k * b, b)])
    last = ch - nck * b
    pltpu.sync_copy(rows2.at[0, pl.ds(0, last)],
                    acc_sh.at[pl.ds(zbase + nck * b, last)])
    pltpu.sync_copy(asrc_v.at[pl.ds(0, ch)], s_sh.at[pl.ds(zbase, ch)])

    @pl.when(tid == NS - 1)
    def _():
        pltpu.sync_copy(rows2.at[0, pl.ds(0, rem)],
                        acc_sh.at[pl.ds(NS * ch, rem)])
        pltpu.sync_copy(asrc_v.at[pl.ds(0, rem)],
                        s_sh.at[pl.ds(NS * ch, rem)])

    pltpu.sync_copy(asrc_hbm, asrc_v)
    pltpu.sync_copy(adst_hbm, adst_v)
    plsc.subcore_barrier()

    def issue_idx(p, off):
        return (pltpu.async_copy(src_hbm.at[pl.ds(off, b)], sidx2.at[p], isem[p]),
                pltpu.async_copy(dst_hbm.at[pl.ds(off, b)], didx2.at[p], isem[p]))

    def issue_gather(p):
        return pltpu.async_copy(h_hbm.at[sidx2.at[p]], rows2.at[p], gsem[p])

    def scores(p):
        for i in range(nl):
            si = sidx2[p, pl.ds(i * LANES, LANES)]
            di = didx2[p, pl.ds(i * LANES, LANES)]
            e = plsc.load_gather(asrc_v, [si]) + plsc.load_gather(adst_v, [di])
            e = jnp.where(e >= 0.0, e, 0.2 * e)
            w2[p, pl.ds(i * LANES, LANES)] = jnp.exp(e)

    def snap_didx(p):
        for i in range(nl):
            dsc2[p, pl.ds(i * LANES, LANES)] = didx2[p, pl.ds(i * LANES, LANES)]

    def scale(p):
        def scale_body(i, c):
            w16 = w2[p, pl.ds(i * LANES, LANES)]
            base = i * LANES
            for j in range(LANES):
                ws = w16[j]
                for d in range(d_out // LANES):
                    sl = rows2[p, base + j, pl.ds(d * LANES, LANES)]
                    rows2[p, base + j, pl.ds(d * LANES, LANES)] = sl * ws
            return c

        lax.fori_loop(0, nl, scale_body, 0)

    def issue_scatter(p):
        return (pltpu.async_copy(rows2.at[p], acc_sh.at[dsc2.at[p]],
                                 ssem[p], add=True),
                pltpu.async_copy(w2.at[p], s_sh.at[dsc2.at[p]],
                                 ssem[p], add=True))

    def quad_body(m, carry):
        b0 = m * 4
        offs = [ebase + (b0 + k) * b for k in range(4)]
        i0 = issue_idx(0, offs[0])
        i1 = issue_idx(1, offs[1])
        i0[0].wait(); i0[1].wait()
        g0 = issue_gather(0)
        scores(0)
        i1[0].wait(); i1[1].wait()
        g0.wait()
        g1 = issue_gather(1)
        i2 = issue_idx(0, offs[2])
        snap_didx(0)
        scale(0)
        s0 = issue_scatter(0)
        scores(1)
        g1.wait()
        i2[0].wait(); i2[1].wait()
        s0[0].wait(); s0[1].wait()
        g2 = issue_gather(0)
        i3 = issue_idx(1, offs[3])
        snap_didx(1)
        scale(1)
        s1 = issue_scatter(1)
        scores(0)
        g2.wait()
        i3[0].wait(); i3[1].wait()
        s1[0].wait(); s1[1].wait()
        g3 = issue_gather(1)
        snap_didx(0)
        scale(0)
        s2 = issue_scatter(0)
        scores(1)
        g3.wait()
        snap_didx(1)
        scale(1)
        s3 = issue_scatter(1)
        s2[0].wait(); s2[1].wait()
        s3[0].wait(); s3[1].wait()
        return carry

    lax.fori_loop(0, nfull // 4, quad_body, 0)

    toff = ebase + nfull * b
    pltpu.sync_copy(src_hbm.at[pl.ds(toff, tail)], tsidx)
    pltpu.sync_copy(dst_hbm.at[pl.ds(toff, tail)], tdidx)
    pltpu.async_copy(h_hbm.at[tsidx], rows2.at[0, pl.ds(0, tail)], gsem0).wait()
    si = tsidx[...]
    di = tdidx[...]
    e = plsc.load_gather(asrc_v, [si]) + plsc.load_gather(adst_v, [di])
    e = jnp.where(e >= 0.0, e, 0.2 * e)
    w2[0, pl.ds(0, tail)] = jnp.exp(e)
    w16 = w2[0, pl.ds(0, tail)]
    for j in range(tail):
        ws = w16[j]
        for d in range(d_out // LANES):
            sl = rows2[0, j, pl.ds(d * LANES, LANES)]
            rows2[0, j, pl.ds(d * LANES, LANES)] = sl * ws
    pltpu.sync_copy(rows2.at[0, pl.ds(0, tail)], acc_sh.at[tdidx], add=True)
    pltpu.sync_copy(w2.at[0, pl.ds(0, tail)], s_sh.at[tdidx], add=True)

    plsc.subcore_barrier()

    for k in range(nck):
        pltpu.sync_copy(acc_sh.at[pl.ds(zbase + k * b, b)], rows2.at[0])
        pltpu.sync_copy(rows2.at[0], acc_out.at[cid, pl.ds(zbase + k * b, b)])
    pltpu.sync_copy(acc_sh.at[pl.ds(zbase + nck * b, last)],
                    rows2.at[0, pl.ds(0, last)])
    pltpu.sync_copy(rows2.at[0, pl.ds(0, last)],
                    acc_out.at[cid, pl.ds(zbase + nck * b, last)])
    pltpu.sync_copy(s_sh.at[pl.ds(zbase, ch)], asrc_v.at[pl.ds(0, ch)])
    pltpu.sync_copy(asrc_v.at[pl.ds(0, ch)],
                    s_out.at[pl.ds(cid * n + zbase, ch)])

    @pl.when(tid == NS - 1)
    def _():
        pltpu.sync_copy(acc_sh.at[pl.ds(NS * ch, rem)],
                        rows2.at[0, pl.ds(0, rem)])
        pltpu.sync_copy(rows2.at[0, pl.ds(0, rem)],
                        acc_out.at[cid, pl.ds(NS * ch, rem)])
        pltpu.sync_copy(s_sh.at[pl.ds(NS * ch, rem)],
                        asrc_v.at[pl.ds(0, rem)])
        pltpu.sync_copy(asrc_v.at[pl.ds(0, rem)],
                        s_out.at[pl.ds(cid * n + NS * ch, rem)])


def kernel(node_features, edge_index, W, attn, bias):
    n, d_in = node_features.shape
    d_out = W.shape[1]
    e_total = edge_index.shape[1]

    a1 = attn[:d_out].reshape(1, d_out)
    a2 = attn[d_out:].reshape(1, d_out)

    h, alpha_src, alpha_dst = pl.pallas_call(
        _pre_body,
        grid=(n // ROWBLK,),
        in_specs=[
            pl.BlockSpec((ROWBLK, d_in), lambda i: (i, 0)),
            pl.BlockSpec((d_in, d_out), lambda i: (0, 0)),
            pl.BlockSpec((1, d_out), lambda i: (0, 0)),
            pl.BlockSpec((1, d_out), lambda i: (0, 0)),
        ],
        out_specs=[
            pl.BlockSpec((ROWBLK, d_out), lambda i: (i, 0)),
            pl.BlockSpec((ROWBLK, 1), lambda i: (i, 0)),
            pl.BlockSpec((ROWBLK, 1), lambda i: (i, 0)),
        ],
        out_shape=[
            jax.ShapeDtypeStruct((n, d_out), jnp.float32),
            jax.ShapeDtypeStruct((n, 1), jnp.float32),
            jax.ShapeDtypeStruct((n, 1), jnp.float32),
        ],
    )(node_features, W, a1, a2)

    src = edge_index[0]
    dst = edge_index[1]

    mesh = plsc.VectorSubcoreMesh(
        core_axis_name="c", subcore_axis_name="s",
        num_cores=NC, num_subcores=NS)
    acc, s = pl.kernel(
        _sc_body,
        out_type=[
            jax.ShapeDtypeStruct((NC, n, d_out), jnp.float32),
            jax.ShapeDtypeStruct((NC * n,), jnp.float32),
        ],
        mesh=mesh,
        compiler_params=pltpu.CompilerParams(needs_layout_passes=False),
        scratch_types=[
            pltpu.VMEM((n,), jnp.float32),
            pltpu.VMEM((n,), jnp.float32),
            pltpu.VMEM((2, BATCH), jnp.int32),
            pltpu.VMEM((2, BATCH), jnp.int32),
            pltpu.VMEM((2, BATCH), jnp.int32),
            pltpu.VMEM((2, BATCH, d_out), jnp.float32),
            pltpu.VMEM((2, BATCH), jnp.float32),
            pltpu.VMEM((LANES,), jnp.int32),
            pltpu.VMEM((LANES,), jnp.int32),
            pltpu.VMEM_SHARED((n, d_out), jnp.float32),
            pltpu.VMEM_SHARED((n,), jnp.float32),
            pltpu.SemaphoreType.DMA,
            pltpu.SemaphoreType.DMA,
            pltpu.SemaphoreType.DMA,
            pltpu.SemaphoreType.DMA,
            pltpu.SemaphoreType.DMA,
            pltpu.SemaphoreType.DMA,
        ],
    )(src, dst, alpha_src.reshape(n), alpha_dst.reshape(n), h)

    out = pl.pallas_call(
        _post_body,
        grid=(n // ROWBLK,),
        in_specs=[
            pl.BlockSpec((NC, ROWBLK, d_out), lambda i: (0, i, 0)),
            pl.BlockSpec((NC, ROWBLK, 1), lambda i: (0, i, 0)),
            pl.BlockSpec((1, d_out), lambda i: (0, 0)),
        ],
        out_specs=pl.BlockSpec((ROWBLK, d_out), lambda i: (i, 0)),
        out_shape=jax.ShapeDtypeStruct((n, d_out), jnp.float32),
    )(acc, s.reshape(NC, n, 1), bias.reshape(1, d_out))
    return out


# speedup vs baseline: 29.6071x; 1.0007x over previous
import functools

import jax
import jax.numpy as jnp
from jax import lax
from jax.experimental import pallas as pl
from jax.experimental.pallas import tpu as pltpu
from jax.experimental.pallas import tpu_sc as plsc

NC = 2
NS = 16
LANES = 16
BATCH = 96
ROWBLK = 2000


def _pre_body(x_ref, w_ref, a1_ref, a2_ref, h_ref, as_ref, ad_ref):
    h = lax.dot_general(
        x_ref[...], w_ref[...], (((1,), (0,)), ((), ())),
        precision=lax.Precision.HIGHEST, preferred_element_type=jnp.float32)
    h_ref[...] = h
    as_ref[...] = jnp.sum(h * a1_ref[...], axis=1, keepdims=True)
    ad_ref[...] = jnp.sum(h * a2_ref[...], axis=1, keepdims=True)


def _post_body(acc_ref, s_ref, b_ref, o_ref):
    acc = acc_ref[0] + acc_ref[1]
    s = s_ref[0] + s_ref[1]
    ok = s > 0.0
    safe = jnp.where(ok, s, 1.0)
    o_ref[...] = jnp.where(ok, acc / safe, 0.0) + b_ref[...]


def _sc_body(src_hbm, dst_hbm, asrc_hbm, adst_hbm, h_hbm,
             acc_out, s_out,
             asrc_v, adst_v, sidx2, didx2, dsc2, rows2, w2, tsidx, tdidx,
             acc_sh, s_sh,
             isem0, isem1, gsem0, gsem1, ssem0, ssem1):
    n = asrc_v.shape[0]
    d_out = rows2.shape[2]
    b = rows2.shape[1]
    nl = b // LANES
    cid = lax.axis_index("c")
    tid = lax.axis_index("s")
    gtid = cid * NS + tid
    e_total = src_hbm.shape[0]
    ept = e_total // (NC * NS)
    nfull = ept // b
    tail = ept - nfull * b
    ebase = gtid * ept
    isem = (isem0, isem1)
    gsem = (gsem0, gsem1)
    ssem = (ssem0, ssem1)

    ch = (n // NS) & ~7
    rem = n - NS * ch
    zbase = tid * ch
    z16 = jnp.zeros((LANES,), jnp.float32)

    def zrow_body(j, c):
        for d in range(d_out // LANES):
            rows2[0, j, pl.ds(d * LANES, LANES)] = z16
        return c

    lax.fori_loop(0, b, zrow_body, 0)

    def zs_body(i, c):
        asrc_v[pl.ds(i * LANES, LANES)] = z16
        return c

    lax.fori_loop(0, ch // LANES, zs_body, 0)

    nck = ch // b
    for k in range(nck):
        pltpu.sync_copy(rows2.at[0], acc_sh.at[pl.ds(zbase + k * b, b)])
    last = ch - nck * b
    pltpu.sync_copy(rows2.at[0, pl.ds(0, last)],
                    acc_sh.at[pl.ds(zbase + nck * b, last)])
    pltpu.sync_copy(asrc_v.at[pl.ds(0, ch)], s_sh.at[pl.ds(zbase, ch)])

    @pl.when(tid == NS - 1)
    def _():
        pltpu.sync_copy(rows2.at[0, pl.ds(0, rem)],
                        acc_sh.at[pl.ds(NS * ch, rem)])
        pltpu.sync_copy(asrc_v.at[pl.ds(0, rem)],
                        s_sh.at[pl.ds(NS * ch, rem)])

    pltpu.sync_copy(asrc_hbm, asrc_v)
    pltpu.sync_copy(adst_hbm, adst_v)
    plsc.subcore_barrier()

    def issue_idx(p, off):
        return (pltpu.async_copy(src_hbm.at[pl.ds(off, b)], sidx2.at[p], isem[p]),
                pltpu.async_copy(dst_hbm.at[pl.ds(off, b)], didx2.at[p], isem[p]))

    def issue_gather(p):
        return pltpu.async_copy(h_hbm.at[sidx2.at[p]], rows2.at[p], gsem[p])

    def scores(p):
        for i in range(nl):
            si = sidx2[p, pl.ds(i * LANES, LANES)]
            di = didx2[p, pl.ds(i * LANES, LANES)]
            e = plsc.load_gather(asrc_v, [si]) + plsc.load_gather(adst_v, [di])
            e = jnp.where(e >= 0.0, e, 0.2 * e)
            w2[p, pl.ds(i * LANES, LANES)] = jnp.exp(e)

    def snap_didx(p):
        for i in range(nl):
            dsc2[p, pl.ds(i * LANES, LANES)] = didx2[p, pl.ds(i * LANES, LANES)]

    def scale(p):
        def scale_body(i, c):
            w16 = w2[p, pl.ds(i * LANES, LANES)]
            base = i * LANES
            for j in range(LANES):
                ws = w16[j]
                for d in range(d_out // LANES):
                    sl = rows2[p, base + j, pl.ds(d * LANES, LANES)]
                    rows2[p, base + j, pl.ds(d * LANES, LANES)] = sl * ws
            return c

        lax.fori_loop(0, nl, scale_body, 0)

    def issue_scatter(p):
        return (pltpu.async_copy(rows2.at[p], acc_sh.at[dsc2.at[p]],
                                 ssem[p], add=True),
                pltpu.async_copy(w2.at[p], s_sh.at[dsc2.at[p]],
                                 ssem[p], add=True))

    def drain_gather(p):
        pltpu.make_async_copy(h_hbm.at[pl.ds(0, b)], rows2.at[p],
                              gsem[p]).wait()

    def drain_scatter(p):
        pltpu.make_async_copy(h_hbm.at[pl.ds(0, b)], rows2.at[p],
                              ssem[p]).wait()
        pltpu.make_async_copy(src_hbm.at[pl.ds(0, b)], dsc2.at[p],
                              ssem[p]).wait()

    def quad_body(m, carry):
        b0 = m * 4
        offs = [ebase + (b0 + k) * b for k in range(4)]
        i0 = issue_idx(0, offs[0])
        i1 = issue_idx(1, offs[1])
        i0[0].wait(); i0[1].wait()
        issue_gather(0)
        scores(0)
        i1[0].wait(); i1[1].wait()
        drain_gather(0)
        issue_gather(1)
        i2 = issue_idx(0, offs[2])
        snap_didx(0)
        scale(0)
        issue_scatter(0)
        scores(1)
        drain_gather(1)
        i2[0].wait(); i2[1].wait()
        drain_scatter(0)
        issue_gather(0)
        i3 = issue_idx(1, offs[3])
        snap_didx(1)
        scale(1)
        issue_scatter(1)
        scores(0)
        drain_gather(0)
        i3[0].wait(); i3[1].wait()
        drain_scatter(1)
        issue_gather(1)
        snap_didx(0)
        scale(0)
        issue_scatter(0)
        scores(1)
        drain_gather(1)
        snap_didx(1)
        scale(1)
        issue_scatter(1)
        drain_scatter(0)
        drain_scatter(1)
        return carry

    lax.fori_loop(0, nfull // 4, quad_body, 0)

    toff = ebase + nfull * b
    pltpu.sync_copy(src_hbm.at[pl.ds(toff, tail)], tsidx)
    pltpu.sync_copy(dst_hbm.at[pl.ds(toff, tail)], tdidx)
    pltpu.async_copy(h_hbm.at[tsidx], rows2.at[0, pl.ds(0, tail)], gsem0).wait()
    si = tsidx[...]
    di = tdidx[...]
    e = plsc.load_gather(asrc_v, [si]) + plsc.load_gather(adst_v, [di])
    e = jnp.where(e >= 0.0, e, 0.2 * e)
    w2[0, pl.ds(0, tail)] = jnp.exp(e)
    w16 = w2[0, pl.ds(0, tail)]
    for j in range(tail):
        ws = w16[j]
        for d in range(d_out // LANES):
            sl = rows2[0, j, pl.ds(d * LANES, LANES)]
            rows2[0, j, pl.ds(d * LANES, LANES)] = sl * ws
    pltpu.sync_copy(rows2.at[0, pl.ds(0, tail)], acc_sh.at[tdidx], add=True)
    pltpu.sync_copy(w2.at[0, pl.ds(0, tail)], s_sh.at[tdidx], add=True)

    plsc.subcore_barrier()

    for k in range(nck):
        pltpu.sync_copy(acc_sh.at[pl.ds(zbase + k * b, b)], rows2.at[0])
        pltpu.sync_copy(rows2.at[0], acc_out.at[cid, pl.ds(zbase + k * b, b)])
    pltpu.sync_copy(acc_sh.at[pl.ds(zbase + nck * b, last)],
                    rows2.at[0, pl.ds(0, last)])
    pltpu.sync_copy(rows2.at[0, pl.ds(0, last)],
                    acc_out.at[cid, pl.ds(zbase + nck * b, last)])
    pltpu.sync_copy(s_sh.at[pl.ds(zbase, ch)], asrc_v.at[pl.ds(0, ch)])
    pltpu.sync_copy(asrc_v.at[pl.ds(0, ch)],
                    s_out.at[pl.ds(cid * n + zbase, ch)])

    @pl.when(tid == NS - 1)
    def _():
        pltpu.sync_copy(acc_sh.at[pl.ds(NS * ch, rem)],
                        rows2.at[0, pl.ds(0, rem)])
        pltpu.sync_copy(rows2.at[0, pl.ds(0, rem)],
                        acc_out.at[cid, pl.ds(NS * ch, rem)])
        pltpu.sync_copy(s_sh.at[pl.ds(NS * ch, rem)],
                        asrc_v.at[pl.ds(0, rem)])
        pltpu.sync_copy(asrc_v.at[pl.ds(0, rem)],
                        s_out.at[pl.ds(cid * n + NS * ch, rem)])


def kernel(node_features, edge_index, W, attn, bias):
    n, d_in = node_features.shape
    d_out = W.shape[1]
    e_total = edge_index.shape[1]

    a1 = attn[:d_out].reshape(1, d_out)
    a2 = attn[d_out:].reshape(1, d_out)

    h, alpha_src, alpha_dst = pl.pallas_call(
        _pre_body,
        grid=(n // ROWBLK,),
        in_specs=[
            pl.BlockSpec((ROWBLK, d_in), lambda i: (i, 0)),
            pl.BlockSpec((d_in, d_out), lambda i: (0, 0)),
            pl.BlockSpec((1, d_out), lambda i: (0, 0)),
            pl.BlockSpec((1, d_out), lambda i: (0, 0)),
        ],
        out_specs=[
            pl.BlockSpec((ROWBLK, d_out), lambda i: (i, 0)),
            pl.BlockSpec((ROWBLK, 1), lambda i: (i, 0)),
            pl.BlockSpec((ROWBLK, 1), lambda i: (i, 0)),
        ],
        out_shape=[
            jax.ShapeDtypeStruct((n, d_out), jnp.float32),
            jax.ShapeDtypeStruct((n, 1), jnp.float32),
            jax.ShapeDtypeStruct((n, 1), jnp.float32),
        ],
    )(node_features, W, a1, a2)

    src = edge_index[0]
    dst = edge_index[1]

    mesh = plsc.VectorSubcoreMesh(
        core_axis_name="c", subcore_axis_name="s",
        num_cores=NC, num_subcores=NS)
    acc, s = pl.kernel(
        _sc_body,
        out_type=[
            jax.ShapeDtypeStruct((NC, n, d_out), jnp.float32),
            jax.ShapeDtypeStruct((NC * n,), jnp.float32),
        ],
        mesh=mesh,
        compiler_params=pltpu.CompilerParams(needs_layout_passes=False),
        scratch_types=[
            pltpu.VMEM((n,), jnp.float32),
            pltpu.VMEM((n,), jnp.float32),
            pltpu.VMEM((2, BATCH), jnp.int32),
            pltpu.VMEM((2, BATCH), jnp.int32),
            pltpu.VMEM((2, BATCH), jnp.int32),
            pltpu.VMEM((2, BATCH, d_out), jnp.float32),
            pltpu.VMEM((2, BATCH), jnp.float32),
            pltpu.VMEM((LANES,), jnp.int32),
            pltpu.VMEM((LANES,), jnp.int32),
            pltpu.VMEM_SHARED((n, d_out), jnp.float32),
            pltpu.VMEM_SHARED((n,), jnp.float32),
            pltpu.SemaphoreType.DMA,
            pltpu.SemaphoreType.DMA,
            pltpu.SemaphoreType.DMA,
            pltpu.SemaphoreType.DMA,
            pltpu.SemaphoreType.DMA,
            pltpu.SemaphoreType.DMA,
        ],
    )(src, dst, alpha_src.reshape(n), alpha_dst.reshape(n), h)

    out = pl.pallas_call(
        _post_body,
        grid=(n // ROWBLK,),
        in_specs=[
            pl.BlockSpec((NC, ROWBLK, d_out), lambda i: (0, i, 0)),
            pl.BlockSpec((NC, ROWBLK, 1), lambda i: (0, i, 0)),
            pl.BlockSpec((1, d_out), lambda i: (0, 0)),
        ],
        out_specs=pl.BlockSpec((ROWBLK, d_out), lambda i: (i, 0)),
        out_shape=jax.ShapeDtypeStruct((n, d_out), jnp.float32),
    )(acc, s.reshape(NC, n, 1), bias.reshape(1, d_out))
    return out
